# baseline (speedup 1.0000x reference)
import functools

import jax
import jax.numpy as jnp
from jax import lax
from jax.experimental import pallas as pl
from jax.experimental.pallas import tpu as pltpu
from jax.experimental.pallas import tpu_sc as plsc

_SUBRATE = 0.1
_VB = 1000
_SEG = 25000
_NWORKERS = 32


def _gather_rows(item_embs, qids):
    nq = qids.shape[0]
    v, d = item_embs.shape
    per_w = nq // _NWORKERS
    mesh = plsc.VectorSubcoreMesh(core_axis_name="c", subcore_axis_name="s")

    @functools.partial(
        pl.kernel,
        mesh=mesh,
        compiler_params=pltpu.CompilerParams(use_tc_tiling_on_sc=False),
        out_type=jax.ShapeDtypeStruct((nq, d), jnp.float32),
        scratch_types=[
            pltpu.VMEM((per_w,), jnp.int32),
            pltpu.VMEM((per_w, d), jnp.float32),
            pltpu.SemaphoreType.DMA,
        ],
    )
    def gk(table_hbm, idx_hbm, out_hbm, idx_v, rows_v, sem):
        wid = lax.axis_index("s") * 2 + lax.axis_index("c")
        base = wid * per_w
        pltpu.sync_copy(idx_hbm.at[pl.ds(base, per_w)], idx_v)
        pltpu.async_copy(table_hbm.at[idx_v], rows_v, sem).wait()
        pltpu.sync_copy(rows_v, out_hbm.at[pl.ds(base, per_w)])

    return gk(item_embs, qids)


def _top1_body(qt_ref, qids_ref, e_ref, r_ref, pick_ref, segm_ref, segi_ref):
    i = pl.program_id(0)
    vb, nq = e_ref.shape[0], qt_ref.shape[0]
    per_seg = _SEG // vb
    scores = lax.dot_general(
        e_ref[...],
        qt_ref[...],
        (((1,), (1,)), ((), ())),
        preferred_element_type=jnp.float32,
    )
    liota = lax.broadcasted_iota(jnp.int32, (vb, nq), 0)
    qs = qids_ref[...] - i * vb
    masked = jnp.where(liota == qs, -jnp.inf, scores)
    bmax = jnp.max(masked, axis=0, keepdims=True)
    bidx = jnp.argmax(masked, axis=0).astype(jnp.int32)[None, :] + i * vb

    @pl.when(i % per_seg == 0)
    def _():
        segm_ref[...] = bmax
        segi_ref[...] = bidx

    @pl.when(i % per_seg != 0)
    def _():
        better = bmax > segm_ref[...]
        segm_ref[...] = jnp.where(better, bmax, segm_ref[...])
        segi_ref[...] = jnp.where(better, bidx, segi_ref[...])

    @pl.when(i % per_seg == per_seg - 1)
    def _():
        win = segm_ref[...] > r_ref[...]
        q16 = segm_ref[...].astype(jnp.bfloat16).astype(jnp.float32)
        first_seg = i == per_seg - 1
        r_ref[...] = jnp.where(first_seg | win, q16, r_ref[...])
        pick_ref[...] = jnp.where(first_seg | win, segi_ref[...], pick_ref[...])


def _top1_call(qt, qids_row, item_embs_bf16):
    nq, d = qt.shape
    v = item_embs_bf16.shape[0]
    nb = v // _VB
    _, pick = pl.pallas_call(
        _top1_body,
        grid=(nb,),
        in_specs=[
            pl.BlockSpec((nq, d), lambda i: (0, 0)),
            pl.BlockSpec((1, nq), lambda i: (0, 0)),
            pl.BlockSpec((_VB, d), lambda i: (i, 0)),
        ],
        out_specs=[
            pl.BlockSpec((1, nq), lambda i: (0, 0)),
            pl.BlockSpec((1, nq), lambda i: (0, 0)),
        ],
        out_shape=[
            jax.ShapeDtypeStruct((1, nq), jnp.float32),
            jax.ShapeDtypeStruct((1, nq), jnp.int32),
        ],
        scratch_shapes=[
            pltpu.VMEM((1, nq), jnp.float32),
            pltpu.VMEM((1, nq), jnp.int32),
        ],
    )(qt, qids_row, item_embs_bf16)
    return pick


def _scatter_body(seq_ref, pos_ref, sub_ref, nsub_ref, out_ref):
    b, l = seq_ref.shape
    nmax = pos_ref.shape[1]
    lane = lax.broadcasted_iota(jnp.int32, (b, l), 1)
    out = seq_ref[...]
    for j in range(nmax):
        active = (lane == pos_ref[:, j : j + 1]) & (j < nsub_ref[...])
        out = jnp.where(active, sub_ref[:, j : j + 1], out)
    out_ref[...] = out


def _scatter_call(seq, pos, subs, nsub):
    b, l = seq.shape
    nmax = pos.shape[1]
    return pl.pallas_call(
        _scatter_body,
        in_specs=[
            pl.BlockSpec((b, l), lambda: (0, 0)),
            pl.BlockSpec((b, nmax), lambda: (0, 0)),
            pl.BlockSpec((b, nmax), lambda: (0, 0)),
            pl.BlockSpec((b, 1), lambda: (0, 0)),
        ],
        out_specs=pl.BlockSpec((b, l), lambda: (0, 0)),
        out_shape=jax.ShapeDtypeStruct((b, l), jnp.int32),
    )(seq, pos, subs, nsub)


def kernel(sequences, seq_lens, item_embs):
    b, l = sequences.shape
    nmax = max(1, int(_SUBRATE * l))

    sub_len = jnp.maximum(1, (_SUBRATE * seq_lens).astype(jnp.int32))
    r = jax.random.uniform(jax.random.key(42), (b, l))
    valid = jnp.arange(l)[None, :] < seq_lens[:, None]
    r = jnp.where(valid, r, jnp.inf)
    lane = jnp.arange(l, dtype=jnp.int32)[None, :]
    cols = []
    for _ in range(nmax):
        idx = jnp.argmin(r, axis=1).astype(jnp.int32)[:, None]
        cols.append(idx)
        r = jnp.where(lane == idx, jnp.inf, r)
    pos = jnp.concatenate(cols, axis=1)

    seq_i32 = sequences.astype(jnp.int32)
    qids = jnp.take_along_axis(seq_i32, pos, axis=1).reshape(-1)

    q = _gather_rows(item_embs, qids)
    qt = q.astype(jnp.bfloat16)
    ebf = item_embs.astype(jnp.bfloat16)
    pick = _top1_call(qt, qids[None, :], ebf)

    out = _scatter_call(seq_i32, pos, pick.reshape(b, nmax), sub_len[:, None])
    return out.astype(sequences.dtype), seq_lens


# speedup vs baseline: 1.0866x; 1.0866x over previous
import functools

import jax
import jax.numpy as jnp
from jax import lax
from jax.experimental import pallas as pl
from jax.experimental.pallas import tpu as pltpu
from jax.experimental.pallas import tpu_sc as plsc

_SUBRATE = 0.1
_VB = 1000
_SEG = 25000
_NWORKERS = 32


def _gather_rows(item_embs, qids):
    nq = qids.shape[0]
    v, d = item_embs.shape
    per_w = nq // _NWORKERS
    mesh = plsc.VectorSubcoreMesh(core_axis_name="c", subcore_axis_name="s")

    @functools.partial(
        pl.kernel,
        mesh=mesh,
        compiler_params=pltpu.CompilerParams(use_tc_tiling_on_sc=False),
        out_type=jax.ShapeDtypeStruct((nq, d), jnp.float32),
        scratch_types=[
            pltpu.VMEM((per_w,), jnp.int32),
            pltpu.VMEM((per_w, d), jnp.float32),
            pltpu.SemaphoreType.DMA,
        ],
    )
    def gk(table_hbm, idx_hbm, out_hbm, idx_v, rows_v, sem):
        wid = lax.axis_index("s") * 2 + lax.axis_index("c")
        base = wid * per_w
        pltpu.sync_copy(idx_hbm.at[pl.ds(base, per_w)], idx_v)
        pltpu.async_copy(table_hbm.at[idx_v], rows_v, sem).wait()
        pltpu.sync_copy(rows_v, out_hbm.at[pl.ds(base, per_w)])

    return gk(item_embs, qids)


def _top1_body(qt_ref, qids_ref, e_ref, r_ref, pick_ref, segm_ref, segi_ref):
    i = pl.program_id(0)
    vb, nq = e_ref.shape[0], qt_ref.shape[1]
    per_seg = _SEG // vb
    scores = jnp.dot(e_ref[...], qt_ref[...], preferred_element_type=jnp.float32)
    liota = lax.broadcasted_iota(jnp.int32, (vb, nq), 0)
    qs = qids_ref[...] - i * vb
    masked = jnp.where(liota == qs, -jnp.inf, scores)
    bmax = jnp.max(masked, axis=0, keepdims=True)
    bidx = jnp.argmax(masked, axis=0).astype(jnp.int32)[None, :] + i * vb

    @pl.when(i % per_seg == 0)
    def _():
        segm_ref[...] = bmax
        segi_ref[...] = bidx

    @pl.when(i % per_seg != 0)
    def _():
        better = bmax > segm_ref[...]
        segm_ref[...] = jnp.where(better, bmax, segm_ref[...])
        segi_ref[...] = jnp.where(better, bidx, segi_ref[...])

    @pl.when(i % per_seg == per_seg - 1)
    def _():
        win = segm_ref[...] > r_ref[...]
        q16 = segm_ref[...].astype(jnp.bfloat16).astype(jnp.float32)
        first_seg = i == per_seg - 1
        r_ref[...] = jnp.where(first_seg | win, q16, r_ref[...])
        pick_ref[...] = jnp.where(first_seg | win, segi_ref[...], pick_ref[...])


def _top1_call(qt, qids_row, item_embs_bf16):
    d, nq = qt.shape
    v = item_embs_bf16.shape[0]
    nb = v // _VB
    _, pick = pl.pallas_call(
        _top1_body,
        grid=(nb,),
        in_specs=[
            pl.BlockSpec((d, nq), lambda i: (0, 0)),
            pl.BlockSpec((1, nq), lambda i: (0, 0)),
            pl.BlockSpec((_VB, d), lambda i: (i, 0)),
        ],
        out_specs=[
            pl.BlockSpec((1, nq), lambda i: (0, 0)),
            pl.BlockSpec((1, nq), lambda i: (0, 0)),
        ],
        out_shape=[
            jax.ShapeDtypeStruct((1, nq), jnp.float32),
            jax.ShapeDtypeStruct((1, nq), jnp.int32),
        ],
        scratch_shapes=[
            pltpu.VMEM((1, nq), jnp.float32),
            pltpu.VMEM((1, nq), jnp.int32),
        ],
    )(qt, qids_row, item_embs_bf16)
    return pick


def _scatter_body(seq_ref, pos_ref, sub_ref, nsub_ref, out_ref):
    b, l = seq_ref.shape
    nmax = pos_ref.shape[1]
    lane = lax.broadcasted_iota(jnp.int32, (b, l), 1)
    out = seq_ref[...]
    for j in range(nmax):
        active = (lane == pos_ref[:, j : j + 1]) & (j < nsub_ref[...])
        out = jnp.where(active, sub_ref[:, j : j + 1], out)
    out_ref[...] = out


def _scatter_call(seq, pos, subs, nsub):
    b, l = seq.shape
    nmax = pos.shape[1]
    return pl.pallas_call(
        _scatter_body,
        in_specs=[
            pl.BlockSpec((b, l), lambda: (0, 0)),
            pl.BlockSpec((b, nmax), lambda: (0, 0)),
            pl.BlockSpec((b, nmax), lambda: (0, 0)),
            pl.BlockSpec((b, 1), lambda: (0, 0)),
        ],
        out_specs=pl.BlockSpec((b, l), lambda: (0, 0)),
        out_shape=jax.ShapeDtypeStruct((b, l), jnp.int32),
    )(seq, pos, subs, nsub)


def kernel(sequences, seq_lens, item_embs):
    b, l = sequences.shape
    nmax = max(1, int(_SUBRATE * l))

    sub_len = jnp.maximum(1, (_SUBRATE * seq_lens).astype(jnp.int32))
    r = jax.random.uniform(jax.random.key(42), (b, l))
    valid = jnp.arange(l)[None, :] < seq_lens[:, None]
    r = jnp.where(valid, r, jnp.inf)
    order = jnp.argsort(r, axis=1)
    pos = order[:, :nmax].astype(jnp.int32)

    seq_i32 = sequences.astype(jnp.int32)
    qids = jnp.take_along_axis(seq_i32, pos, axis=1).reshape(-1)

    q = _gather_rows(item_embs, qids)
    qt = q.T.astype(jnp.bfloat16)
    ebf = item_embs.astype(jnp.bfloat16)
    pick = _top1_call(qt, qids[None, :], ebf)

    out = _scatter_call(seq_i32, pos, pick.reshape(b, nmax), sub_len[:, None])
    return out.astype(sequences.dtype), seq_lens
